# Initial kernel scaffold; baseline (speedup 1.0000x reference)
import jax
import jax.numpy as jnp
from jax.experimental import pallas as pl

NUM_T = 8192
D_MODEL = 2048
NUM_E = 64
K = 256

BT = 512


def _router_block(x_ref, wt_ref, b_ref, probs_ref):
    logits = jnp.dot(x_ref[...], wt_ref[...],
                     preferred_element_type=jnp.float32) + b_ref[...]
    m = jnp.max(logits, axis=-1, keepdims=True)
    e = jnp.exp(logits - m)
    s = jnp.sum(e, axis=-1, keepdims=True)
    probs_ref[...] = e / s


def kernel(x, W, b):
    wt = W.T
    b2 = b.reshape(1, NUM_E)
    probs = pl.pallas_call(
        _router_block,
        grid=(NUM_T // BT,),
        in_specs=[
            pl.BlockSpec((BT, D_MODEL), lambda i: (i, 0)),
            pl.BlockSpec((D_MODEL, NUM_E), lambda i: (0, 0)),
            pl.BlockSpec((1, NUM_E), lambda i: (0, 0)),
        ],
        out_specs=pl.BlockSpec((BT, NUM_E), lambda i: (i, 0)),
        out_shape=jax.ShapeDtypeStruct((NUM_T, NUM_E), jnp.float32),
    )(x, wt, b2)
    vals, idx = jax.lax.top_k(probs.T, K)
    return vals, idx


# speedup vs baseline: 2.3136x; 2.3136x over previous
import functools

import jax
import jax.numpy as jnp
from jax import lax
from jax.experimental import pallas as pl
from jax.experimental.pallas import tpu as pltpu
from jax.experimental.pallas import tpu_sc as plsc

NUM_T = 8192
D_MODEL = 2048
NUM_E = 64
K = 256

BT = 512
L = 16


def _router_block(x_ref, wt_ref, b_ref, pt_ref):
    x = x_ref[...]
    wt = wt_ref[...]
    acc = jnp.zeros((BT, NUM_E), jnp.float32)
    for k in range(8):
        acc = acc + jnp.dot(x[:, k * 256:(k + 1) * 256],
                            wt[k * 256:(k + 1) * 256, :],
                            preferred_element_type=jnp.float32)
    logits = acc + b_ref[...]
    m = jnp.max(logits, axis=-1, keepdims=True)
    e = jnp.exp(logits - m)
    a8 = e[:, 0:8]
    for r in range(1, 8):
        a8 = a8 + e[:, r * 8:(r + 1) * 8]
    a4 = a8[:, 0:4] + a8[:, 4:8]
    a2 = a4[:, 0:2] + a4[:, 2:4]
    s = a2[:, 0:1] + a2[:, 1:2]
    pt_ref[...] = (e / s).T


def _probs_t(x, W, b):
    wt = W.T
    b2 = b.reshape(1, NUM_E)
    return pl.pallas_call(
        _router_block,
        grid=(NUM_T // BT,),
        in_specs=[
            pl.BlockSpec((BT, D_MODEL), lambda i: (i, 0)),
            pl.BlockSpec((D_MODEL, NUM_E), lambda i: (0, 0)),
            pl.BlockSpec((1, NUM_E), lambda i: (0, 0)),
        ],
        out_specs=pl.BlockSpec((NUM_E, BT), lambda i: (0, i)),
        out_shape=jax.ShapeDtypeStruct((NUM_E, NUM_T), jnp.float32),
    )(x, wt, b2)


def _splat(v):
    return jnp.full((L,), v, jnp.int32)


def _topk_body(probs_hbm, vals_hbm, idx_hbm,
               rowbuf, hist_sel, hist_sort,
               key_a, idx_a, key_b, idx_b,
               valout, idxout, scr_a, scr_b):
    wid = lax.axis_index("s") * 2 + lax.axis_index("c")
    lanes = lax.iota(jnp.int32, 16)
    ones = jnp.ones((L,), jnp.int32)
    zeros = jnp.zeros((L,), jnp.int32)
    n_vec = NUM_T // L

    def zero_hist_sel():
        def z(i, c):
            hist_sel[pl.ds(i * L, L)] = zeros
            return c
        lax.fori_loop(0, 256, z, 0)

    def find_threshold(target, digit_of, gate_of):
        zero_hist_sel()

        def hbody(j, c):
            bits = rowbuf[pl.ds(j * L, L)]
            dig = digit_of(bits)
            plsc.addupdate_scatter(hist_sel, [lanes * 256 + dig], ones,
                                   mask=gate_of(bits))
            return c
        lax.fori_loop(0, n_vec, hbody, 0)

        def sbody(i, carry):
            found, tsel, ngt, cum = carry
            c = 15 - i
            tot = zeros
            for l in range(16):
                tot = tot + hist_sel[pl.ds(l * 256 + c * L, L)]
            rt = lax.rev(tot, (0,))
            cs = plsc.cumsum(rt) + cum
            mask = cs >= target
            npop = plsc.all_reduce_population_count(mask)
            any_ = npop > 0
            pos = jnp.where(any_, plsc.all_reduce_ffs(mask), 0)
            scr_a[...] = cs
            scr_b[...] = rt
            cum_at = plsc.load_gather(scr_a, [pos])
            cnt_at = plsc.load_gather(scr_b, [pos])
            upd = jnp.logical_and(found == 0, any_)
            tsel = jnp.where(upd, c * L + 15 - pos, tsel)
            ngt = jnp.where(upd, cum_at - cnt_at, ngt)
            found = jnp.where(upd, ones, found)
            cum_out = plsc.load_gather(scr_a, [_splat(15)])
            return found, tsel, ngt, cum_out

        _, t, ngt, _ = lax.fori_loop(
            0, 16, sbody, (zeros, zeros, zeros, zeros))
        return t, ngt

    for r in range(2):
        row = wid * 2 + r
        pltpu.sync_copy(probs_hbm.at[row], rowbuf)

        t1, ngt1 = find_threshold(
            _splat(K),
            lambda bits: lax.shift_right_logical(bits, 24),
            lambda bits: jnp.ones((L,), jnp.bool_))
        t2, _ = find_threshold(
            _splat(K) - ngt1,
            lambda bits: jnp.bitwise_and(lax.shift_right_logical(bits, 16), 255),
            lambda bits: lax.shift_right_logical(bits, 24) == t1)
        pref = t1 * 256 + t2

        def cbody(j, off):
            bits = rowbuf[pl.ds(j * L, L)]
            msk = lax.shift_right_logical(bits, 16) >= pref
            mi = msk.astype(jnp.int32)
            incl = plsc.cumsum(mi)
            posn = off + incl - mi
            plsc.store_scatter(key_a, [posn], bits, mask=msk)
            plsc.store_scatter(idx_a, [posn], j * L + lanes, mask=msk)
            return off + plsc.all_reduce_population_count(msk)
        m_cnt = lax.fori_loop(0, n_vec, cbody, zeros)

        plsc.store_scatter(key_a, [m_cnt + lanes], zeros)
        plsc.store_scatter(idx_a, [m_cnt + lanes], zeros)

        b_vec = lax.shift_right_logical(m_cnt + 15, 4)
        b_sc = lax.squeeze(lax.slice(b_vec, (0,), (1,)), (0,))

        bufs = [(key_a, idx_a, key_b, idx_b), (key_b, idx_b, key_a, idx_a)]
        for p in range(8):
            shift = p * 4
            src_k, src_i, dst_k, dst_i = bufs[p % 2]
            for c in range(16):
                hist_sort[pl.ds(c * L, L)] = zeros

            def p1(j, carry):
                addr = lanes * b_sc + j
                kv = plsc.load_gather(src_k, [addr])
                dig = 15 - jnp.bitwise_and(
                    lax.shift_right_logical(kv, shift), 15)
                plsc.addupdate_scatter(hist_sort, [dig * L + lanes], ones)
                return carry
            lax.fori_loop(0, b_sc, p1, 0)

            def p2(c, carry):
                v = hist_sort[pl.ds(c * L, L)]
                incl = plsc.cumsum(v)
                hist_sort[pl.ds(c * L, L)] = incl - v + carry
                scr_a[...] = incl
                return carry + plsc.load_gather(scr_a, [_splat(15)])
            lax.fori_loop(0, 16, p2, zeros)

            def p3(j, carry):
                addr = lanes * b_sc + j
                kv = plsc.load_gather(src_k, [addr])
                iv = plsc.load_gather(src_i, [addr])
                dig = 15 - jnp.bitwise_and(
                    lax.shift_right_logical(kv, shift), 15)
                slot = dig * L + lanes
                off = plsc.load_gather(hist_sort, [slot])
                plsc.store_scatter(dst_k, [off], kv)
                plsc.store_scatter(dst_i, [off], iv)
                plsc.store_scatter(hist_sort, [slot], off + 1)
                return carry
            lax.fori_loop(0, b_sc, p3, 0)

        for c in range(K // L):
            valout[pl.ds(c * L, L)] = key_a[pl.ds(c * L, L)]
            idxout[pl.ds(c * L, L)] = idx_a[pl.ds(c * L, L)]
        pltpu.sync_copy(valout, vals_hbm.at[row])
        pltpu.sync_copy(idxout, idx_hbm.at[row])


def _topk_sc(probs_t):
    mesh = plsc.VectorSubcoreMesh(core_axis_name="c", subcore_axis_name="s")
    fn = functools.partial(
        pl.kernel,
        mesh=mesh,
        compiler_params=pltpu.CompilerParams(needs_layout_passes=False),
        out_type=[
            jax.ShapeDtypeStruct((NUM_E, K), jnp.int32),
            jax.ShapeDtypeStruct((NUM_E, K), jnp.int32),
        ],
        scratch_types=[
            pltpu.VMEM((NUM_T,), jnp.int32),
            pltpu.VMEM((4096,), jnp.int32),
            pltpu.VMEM((256,), jnp.int32),
            pltpu.VMEM((NUM_T + L,), jnp.int32),
            pltpu.VMEM((NUM_T + L,), jnp.int32),
            pltpu.VMEM((NUM_T + L,), jnp.int32),
            pltpu.VMEM((NUM_T + L,), jnp.int32),
            pltpu.VMEM((K,), jnp.int32),
            pltpu.VMEM((K,), jnp.int32),
            pltpu.VMEM((L,), jnp.int32),
            pltpu.VMEM((L,), jnp.int32),
        ],
    )(_topk_body)
    return fn(probs_t)


def kernel(x, W, b):
    probs_t = _probs_t(x, W, b)
    key_bits, idx = _topk_sc(lax.bitcast_convert_type(probs_t, jnp.int32))
    return lax.bitcast_convert_type(key_bits, jnp.float32), idx


# speedup vs baseline: 2.3995x; 1.0371x over previous
import functools

import jax
import jax.numpy as jnp
from jax import lax
from jax.experimental import pallas as pl
from jax.experimental.pallas import tpu as pltpu
from jax.experimental.pallas import tpu_sc as plsc

NUM_T = 8192
D_MODEL = 2048
NUM_E = 64
K = 256

BT = 512
L = 16


def _router_block(x_ref, wt_ref, b_ref, pt_ref):
    x = x_ref[...]
    wt = wt_ref[...]
    acc = jnp.zeros((BT, NUM_E), jnp.float32)
    for k in range(8):
        acc = acc + jnp.dot(x[:, k * 256:(k + 1) * 256],
                            wt[k * 256:(k + 1) * 256, :],
                            preferred_element_type=jnp.float32)
    logits = acc + b_ref[...]
    m = jnp.max(logits, axis=-1, keepdims=True)
    e = jnp.exp(logits - m)
    a8 = e[:, 0:8]
    for r in range(1, 8):
        a8 = a8 + e[:, r * 8:(r + 1) * 8]
    a4 = a8[:, 0:4] + a8[:, 4:8]
    a2 = a4[:, 0:2] + a4[:, 2:4]
    s = a2[:, 0:1] + a2[:, 1:2]
    pt_ref[...] = (e / s).T


def _probs_t(x, W, b):
    wt = W.T
    b2 = b.reshape(1, NUM_E)
    return pl.pallas_call(
        _router_block,
        grid=(NUM_T // BT,),
        in_specs=[
            pl.BlockSpec((BT, D_MODEL), lambda i: (i, 0)),
            pl.BlockSpec((D_MODEL, NUM_E), lambda i: (0, 0)),
            pl.BlockSpec((1, NUM_E), lambda i: (0, 0)),
        ],
        out_specs=pl.BlockSpec((NUM_E, BT), lambda i: (0, i)),
        out_shape=jax.ShapeDtypeStruct((NUM_E, NUM_T), jnp.float32),
    )(x, wt, b2)


def _splat(v):
    return jnp.full((L,), v, jnp.int32)


def _topk_body(probs_hbm, vals_hbm, idx_hbm,
               rowbuf, hist_sel, hist_sort,
               key_a, idx_a, key_b, idx_b,
               valout, idxout, scr_a, scr_b):
    wid = lax.axis_index("s") * 2 + lax.axis_index("c")
    lanes = lax.iota(jnp.int32, 16)
    ones = jnp.ones((L,), jnp.int32)
    zeros = jnp.zeros((L,), jnp.int32)
    n_vec = NUM_T // L

    lane_off = lanes * 256

    def zero_hist_sel():
        def z(i, c):
            for u in range(4):
                hist_sel[pl.ds(i * 4 * L + u * L, L)] = zeros
            return c
        lax.fori_loop(0, 64, z, 0)

    def find_threshold(target, digit_of, gate_of):
        zero_hist_sel()

        def hbody(j, c):
            for u in range(4):
                bits = rowbuf[pl.ds(j * (4 * L) + u * L, L)]
                dig = digit_of(bits)
                mask = None if gate_of is None else gate_of(bits)
                plsc.addupdate_scatter(hist_sel, [lane_off + dig], ones,
                                       mask=mask)
            return c
        lax.fori_loop(0, n_vec // 4, hbody, 0)

        def sbody(i, carry):
            found, tsel, ngt, cum = carry
            c = 15 - i
            tot = zeros
            for l in range(16):
                tot = tot + hist_sel[pl.ds(l * 256 + c * L, L)]
            rt = lax.rev(tot, (0,))
            cs = plsc.cumsum(rt) + cum
            mask = cs >= target
            npop = plsc.all_reduce_population_count(mask)
            any_ = npop > 0
            pos = jnp.where(any_, plsc.all_reduce_ffs(mask), 0)
            scr_a[...] = cs
            scr_b[...] = rt
            cum_at = plsc.load_gather(scr_a, [pos])
            cnt_at = plsc.load_gather(scr_b, [pos])
            upd = jnp.logical_and(found == 0, any_)
            tsel = jnp.where(upd, c * L + 15 - pos, tsel)
            ngt = jnp.where(upd, cum_at - cnt_at, ngt)
            found = jnp.where(upd, ones, found)
            cum_out = plsc.load_gather(scr_a, [_splat(15)])
            return found, tsel, ngt, cum_out

        _, t, ngt, _ = lax.fori_loop(
            0, 16, sbody, (zeros, zeros, zeros, zeros))
        return t, ngt

    for r in range(2):
        row = wid * 2 + r
        pltpu.sync_copy(probs_hbm.at[row], rowbuf)

        t1, ngt1 = find_threshold(
            _splat(K),
            lambda bits: lax.shift_right_logical(bits, 24),
            None)
        t2, _ = find_threshold(
            _splat(K) - ngt1,
            lambda bits: jnp.bitwise_and(lax.shift_right_logical(bits, 16), 255),
            lambda bits: lax.shift_right_logical(bits, 24) == t1)
        pref = t1 * 256 + t2

        def cbody(j, off):
            for u in range(4):
                bits = rowbuf[pl.ds(j * (4 * L) + u * L, L)]
                msk = lax.shift_right_logical(bits, 16) >= pref
                mi = msk.astype(jnp.int32)
                incl = plsc.cumsum(mi)
                posn = off + incl - mi
                plsc.store_scatter(key_a, [posn], bits, mask=msk)
                plsc.store_scatter(idx_a, [posn], j * (4 * L) + u * L + lanes,
                                   mask=msk)
                off = off + plsc.all_reduce_population_count(msk)
            return off
        m_cnt = lax.fori_loop(0, n_vec // 4, cbody, zeros)

        plsc.store_scatter(key_a, [m_cnt + lanes], zeros)
        plsc.store_scatter(idx_a, [m_cnt + lanes], zeros)

        b_vec = lax.shift_right_logical(m_cnt + 15, 4)
        b_sc = lax.squeeze(lax.slice(b_vec, (0,), (1,)), (0,))

        bufs = [(key_a, idx_a, key_b, idx_b), (key_b, idx_b, key_a, idx_a)]
        for p in range(8):
            shift = p * 4
            src_k, src_i, dst_k, dst_i = bufs[p % 2]
            for c in range(16):
                hist_sort[pl.ds(c * L, L)] = zeros

            def p1(j, carry):
                addr = lanes * b_sc + j
                kv = plsc.load_gather(src_k, [addr])
                dig = 15 - jnp.bitwise_and(
                    lax.shift_right_logical(kv, shift), 15)
                plsc.addupdate_scatter(hist_sort, [dig * L + lanes], ones)
                return carry
            lax.fori_loop(0, b_sc, p1, 0)

            def p2(c, carry):
                v = hist_sort[pl.ds(c * L, L)]
                incl = plsc.cumsum(v)
                hist_sort[pl.ds(c * L, L)] = incl - v + carry
                scr_a[...] = incl
                return carry + plsc.load_gather(scr_a, [_splat(15)])
            lax.fori_loop(0, 16, p2, zeros)

            def p3(j, carry):
                addr = lanes * b_sc + j
                kv = plsc.load_gather(src_k, [addr])
                iv = plsc.load_gather(src_i, [addr])
                dig = 15 - jnp.bitwise_and(
                    lax.shift_right_logical(kv, shift), 15)
                slot = dig * L + lanes
                off = plsc.load_gather(hist_sort, [slot])
                plsc.store_scatter(dst_k, [off], kv)
                plsc.store_scatter(dst_i, [off], iv)
                plsc.store_scatter(hist_sort, [slot], off + 1)
                return carry
            lax.fori_loop(0, b_sc, p3, 0)

        for c in range(K // L):
            valout[pl.ds(c * L, L)] = key_a[pl.ds(c * L, L)]
            idxout[pl.ds(c * L, L)] = idx_a[pl.ds(c * L, L)]
        pltpu.sync_copy(valout, vals_hbm.at[row])
        pltpu.sync_copy(idxout, idx_hbm.at[row])


def _topk_sc(probs_t):
    mesh = plsc.VectorSubcoreMesh(core_axis_name="c", subcore_axis_name="s")
    fn = functools.partial(
        pl.kernel,
        mesh=mesh,
        compiler_params=pltpu.CompilerParams(needs_layout_passes=False),
        out_type=[
            jax.ShapeDtypeStruct((NUM_E, K), jnp.int32),
            jax.ShapeDtypeStruct((NUM_E, K), jnp.int32),
        ],
        scratch_types=[
            pltpu.VMEM((NUM_T,), jnp.int32),
            pltpu.VMEM((4096,), jnp.int32),
            pltpu.VMEM((256,), jnp.int32),
            pltpu.VMEM((NUM_T + L,), jnp.int32),
            pltpu.VMEM((NUM_T + L,), jnp.int32),
            pltpu.VMEM((NUM_T + L,), jnp.int32),
            pltpu.VMEM((NUM_T + L,), jnp.int32),
            pltpu.VMEM((K,), jnp.int32),
            pltpu.VMEM((K,), jnp.int32),
            pltpu.VMEM((L,), jnp.int32),
            pltpu.VMEM((L,), jnp.int32),
        ],
    )(_topk_body)
    return fn(probs_t)


def kernel(x, W, b):
    probs_t = _probs_t(x, W, b)
    key_bits, idx = _topk_sc(lax.bitcast_convert_type(probs_t, jnp.int32))
    return lax.bitcast_convert_type(key_bits, jnp.float32), idx


# speedup vs baseline: 2.5597x; 1.0667x over previous
import functools

import jax
import jax.numpy as jnp
from jax import lax
from jax.experimental import pallas as pl
from jax.experimental.pallas import tpu as pltpu
from jax.experimental.pallas import tpu_sc as plsc

NUM_T = 8192
D_MODEL = 2048
NUM_E = 64
K = 256

BT = 512
L = 16


def _router_block(x_ref, wt_ref, b_ref, pt_ref):
    x = x_ref[...]
    wt = wt_ref[...]
    acc = jnp.zeros((BT, NUM_E), jnp.float32)
    for k in range(8):
        acc = acc + jnp.dot(x[:, k * 256:(k + 1) * 256],
                            wt[k * 256:(k + 1) * 256, :],
                            preferred_element_type=jnp.float32)
    logits = acc + b_ref[...]
    m = jnp.max(logits, axis=-1, keepdims=True)
    e = jnp.exp(logits - m)
    a8 = e[:, 0:8]
    for r in range(1, 8):
        a8 = a8 + e[:, r * 8:(r + 1) * 8]
    a4 = a8[:, 0:4] + a8[:, 4:8]
    a2 = a4[:, 0:2] + a4[:, 2:4]
    s = a2[:, 0:1] + a2[:, 1:2]
    pt_ref[...] = (e / s).T


def _probs_t(x, W, b):
    wt = W.T
    b2 = b.reshape(1, NUM_E)
    return pl.pallas_call(
        _router_block,
        grid=(NUM_T // BT,),
        in_specs=[
            pl.BlockSpec((BT, D_MODEL), lambda i: (i, 0)),
            pl.BlockSpec((D_MODEL, NUM_E), lambda i: (0, 0)),
            pl.BlockSpec((1, NUM_E), lambda i: (0, 0)),
        ],
        out_specs=pl.BlockSpec((NUM_E, BT), lambda i: (0, i)),
        out_shape=jax.ShapeDtypeStruct((NUM_E, NUM_T), jnp.float32),
    )(x, wt, b2)


def _splat(v):
    return jnp.full((L,), v, jnp.int32)


def _topk_body(probs_hbm, vals_hbm, idx_hbm,
               rowbuf, hist_sel, hist_sort,
               key_a, idx_a, key_b, idx_b,
               valout, idxout, scr_a, scr_b):
    wid = lax.axis_index("s") * 2 + lax.axis_index("c")
    lanes = lax.iota(jnp.int32, 16)
    ones = jnp.ones((L,), jnp.int32)
    zeros = jnp.zeros((L,), jnp.int32)
    n_vec = NUM_T // L

    lane_off = lanes * 256

    def zero_hist_sel():
        def z(i, c):
            for u in range(4):
                hist_sel[pl.ds(i * 4 * L + u * L, L)] = zeros
            return c
        lax.fori_loop(0, 64, z, 0)

    def find_threshold(target, digit_of, gate_of):
        zero_hist_sel()

        def hbody(j, c):
            for u in range(4):
                bits = rowbuf[pl.ds(j * (4 * L) + u * L, L)]
                dig = digit_of(bits)
                mask = None if gate_of is None else gate_of(bits)
                plsc.addupdate_scatter(hist_sel, [lane_off + dig], ones,
                                       mask=mask)
            return c
        lax.fori_loop(0, n_vec // 4, hbody, 0)

        def sbody(i, carry):
            found, tsel, ngt, cum = carry
            c = 15 - i
            tot = zeros
            for l in range(16):
                tot = tot + hist_sel[pl.ds(l * 256 + c * L, L)]
            rt = lax.rev(tot, (0,))
            cs = plsc.cumsum(rt) + cum
            mask = cs >= target
            npop = plsc.all_reduce_population_count(mask)
            any_ = npop > 0
            pos = jnp.where(any_, plsc.all_reduce_ffs(mask), 0)
            scr_a[...] = cs
            scr_b[...] = rt
            cum_at = plsc.load_gather(scr_a, [pos])
            cnt_at = plsc.load_gather(scr_b, [pos])
            upd = jnp.logical_and(found == 0, any_)
            tsel = jnp.where(upd, c * L + 15 - pos, tsel)
            ngt = jnp.where(upd, cum_at - cnt_at, ngt)
            found = jnp.where(upd, ones, found)
            cum_out = plsc.load_gather(scr_a, [_splat(15)])
            return found, tsel, ngt, cum_out

        _, t, ngt, _ = lax.fori_loop(
            0, 16, sbody, (zeros, zeros, zeros, zeros))
        return t, ngt

    for r in range(2):
        row = wid * 2 + r
        pltpu.sync_copy(probs_hbm.at[row], rowbuf)

        t1, ngt1 = find_threshold(
            _splat(K),
            lambda bits: lax.shift_right_logical(bits, 24),
            None)
        t2, _ = find_threshold(
            _splat(K) - ngt1,
            lambda bits: jnp.bitwise_and(lax.shift_right_logical(bits, 16), 255),
            lambda bits: lax.shift_right_logical(bits, 24) == t1)
        pref = t1 * 256 + t2

        def cbody(j, off):
            bitss, msks, mis, incls = [], [], [], []
            for u in range(4):
                bits = rowbuf[pl.ds(j * (4 * L) + u * L, L)]
                msk = lax.shift_right_logical(bits, 16) >= pref
                mi = msk.astype(jnp.int32)
                bitss.append(bits)
                msks.append(msk)
                mis.append(mi)
                incls.append(plsc.cumsum(mi))
            for u in range(4):
                hist_sort[pl.ds(u * L, L)] = incls[u]
            cnts = [plsc.load_gather(hist_sort, [_splat(u * L + 15)])
                    for u in range(4)]
            for u in range(4):
                posn = off + incls[u] - mis[u]
                plsc.store_scatter(key_a, [posn], bitss[u], mask=msks[u])
                plsc.store_scatter(idx_a, [posn], j * (4 * L) + u * L + lanes,
                                   mask=msks[u])
                off = off + cnts[u]
            return off
        m_cnt = lax.fori_loop(0, n_vec // 4, cbody, zeros)

        plsc.store_scatter(key_a, [m_cnt + lanes], zeros)
        plsc.store_scatter(idx_a, [m_cnt + lanes], zeros)

        b_vec = lax.shift_right_logical(m_cnt + 15, 4)
        b_sc = lax.squeeze(lax.slice(b_vec, (0,), (1,)), (0,))

        bufs = [(key_a, idx_a, key_b, idx_b), (key_b, idx_b, key_a, idx_a)]
        for p in range(8):
            shift = p * 4
            src_k, src_i, dst_k, dst_i = bufs[p % 2]
            for c in range(16):
                hist_sort[pl.ds(c * L, L)] = zeros

            def p1(j, carry):
                addr = lanes * b_sc + j
                kv = plsc.load_gather(src_k, [addr])
                dig = 15 - jnp.bitwise_and(
                    lax.shift_right_logical(kv, shift), 15)
                plsc.addupdate_scatter(hist_sort, [dig * L + lanes], ones)
                return carry
            lax.fori_loop(0, b_sc, p1, 0)

            def p2(c, carry):
                v = hist_sort[pl.ds(c * L, L)]
                incl = plsc.cumsum(v)
                hist_sort[pl.ds(c * L, L)] = incl - v + carry
                scr_a[...] = incl
                return carry + plsc.load_gather(scr_a, [_splat(15)])
            lax.fori_loop(0, 16, p2, zeros)

            def p3(j, carry):
                addr = lanes * b_sc + j
                kv = plsc.load_gather(src_k, [addr])
                iv = plsc.load_gather(src_i, [addr])
                dig = 15 - jnp.bitwise_and(
                    lax.shift_right_logical(kv, shift), 15)
                slot = dig * L + lanes
                off = plsc.load_gather(hist_sort, [slot])
                plsc.store_scatter(dst_k, [off], kv)
                plsc.store_scatter(dst_i, [off], iv)
                plsc.store_scatter(hist_sort, [slot], off + 1)
                return carry
            lax.fori_loop(0, b_sc, p3, 0)

        for c in range(K // L):
            valout[pl.ds(c * L, L)] = key_a[pl.ds(c * L, L)]
            idxout[pl.ds(c * L, L)] = idx_a[pl.ds(c * L, L)]
        pltpu.sync_copy(valout, vals_hbm.at[row])
        pltpu.sync_copy(idxout, idx_hbm.at[row])


def _topk_sc(probs_t):
    mesh = plsc.VectorSubcoreMesh(core_axis_name="c", subcore_axis_name="s")
    fn = functools.partial(
        pl.kernel,
        mesh=mesh,
        compiler_params=pltpu.CompilerParams(needs_layout_passes=False),
        out_type=[
            jax.ShapeDtypeStruct((NUM_E, K), jnp.int32),
            jax.ShapeDtypeStruct((NUM_E, K), jnp.int32),
        ],
        scratch_types=[
            pltpu.VMEM((NUM_T,), jnp.int32),
            pltpu.VMEM((4096,), jnp.int32),
            pltpu.VMEM((256,), jnp.int32),
            pltpu.VMEM((NUM_T + L,), jnp.int32),
            pltpu.VMEM((NUM_T + L,), jnp.int32),
            pltpu.VMEM((NUM_T + L,), jnp.int32),
            pltpu.VMEM((NUM_T + L,), jnp.int32),
            pltpu.VMEM((K,), jnp.int32),
            pltpu.VMEM((K,), jnp.int32),
            pltpu.VMEM((L,), jnp.int32),
            pltpu.VMEM((L,), jnp.int32),
        ],
    )(_topk_body)
    return fn(probs_t)


def kernel(x, W, b):
    probs_t = _probs_t(x, W, b)
    key_bits, idx = _topk_sc(lax.bitcast_convert_type(probs_t, jnp.int32))
    return lax.bitcast_convert_type(key_bits, jnp.float32), idx


# speedup vs baseline: 2.5815x; 1.0086x over previous
import functools

import jax
import jax.numpy as jnp
from jax import lax
from jax.experimental import pallas as pl
from jax.experimental.pallas import tpu as pltpu
from jax.experimental.pallas import tpu_sc as plsc

NUM_T = 8192
D_MODEL = 2048
NUM_E = 64
K = 256

BT = 512
L = 16


def _router_block(x_ref, wt_ref, b_ref, pt_ref):
    x = x_ref[...]
    wt = wt_ref[...]
    acc = jnp.zeros((BT, NUM_E), jnp.float32)
    for k in range(8):
        acc = acc + jnp.dot(x[:, k * 256:(k + 1) * 256],
                            wt[k * 256:(k + 1) * 256, :],
                            preferred_element_type=jnp.float32)
    logits = acc + b_ref[...]
    m = jnp.max(logits, axis=-1, keepdims=True)
    e = jnp.exp(logits - m)
    a8 = e[:, 0:8]
    for r in range(1, 8):
        a8 = a8 + e[:, r * 8:(r + 1) * 8]
    a4 = a8[:, 0:4] + a8[:, 4:8]
    a2 = a4[:, 0:2] + a4[:, 2:4]
    s = a2[:, 0:1] + a2[:, 1:2]
    pt_ref[...] = (e / s).T


def _probs_t(x, W, b):
    wt = W.T
    b2 = b.reshape(1, NUM_E)
    return pl.pallas_call(
        _router_block,
        grid=(NUM_T // BT,),
        in_specs=[
            pl.BlockSpec((BT, D_MODEL), lambda i: (i, 0)),
            pl.BlockSpec((D_MODEL, NUM_E), lambda i: (0, 0)),
            pl.BlockSpec((1, NUM_E), lambda i: (0, 0)),
        ],
        out_specs=pl.BlockSpec((NUM_E, BT), lambda i: (0, i)),
        out_shape=jax.ShapeDtypeStruct((NUM_E, NUM_T), jnp.float32),
    )(x, wt, b2)


def _splat(v):
    return jnp.full((L,), v, jnp.int32)


def _topk_body(probs_hbm, vals_hbm, idx_hbm,
               rowbuf, rowbuf2, hist_sel, hist_sort,
               key_a, idx_a, key_b, idx_b,
               valout, idxout, scr_a, scr_b, sem0, sem1):
    wid = lax.axis_index("s") * 2 + lax.axis_index("c")
    lanes = lax.iota(jnp.int32, 16)
    ones = jnp.ones((L,), jnp.int32)
    zeros = jnp.zeros((L,), jnp.int32)
    n_vec = NUM_T // L

    lane_off = lanes * 256

    def zero_hist_sel():
        def z(i, c):
            for u in range(4):
                hist_sel[pl.ds(i * 4 * L + u * L, L)] = zeros
            return c
        lax.fori_loop(0, 64, z, 0)

    def find_threshold(rb, target, digit_of, gate_of):
        zero_hist_sel()

        def hbody(j, c):
            for u in range(4):
                bits = rb[pl.ds(j * (4 * L) + u * L, L)]
                dig = digit_of(bits)
                mask = None if gate_of is None else gate_of(bits)
                plsc.addupdate_scatter(hist_sel, [lane_off + dig], ones,
                                       mask=mask)
            return c
        lax.fori_loop(0, n_vec // 4, hbody, 0)

        def sbody(i, carry):
            found, tsel, ngt, cum = carry
            c = 15 - i
            tot = zeros
            for l in range(16):
                tot = tot + hist_sel[pl.ds(l * 256 + c * L, L)]
            rt = lax.rev(tot, (0,))
            cs = plsc.cumsum(rt) + cum
            mask = cs >= target
            scr_a[...] = cs
            scr_b[...] = rt
            cum_out = plsc.load_gather(scr_a, [_splat(15)])
            any_ = cum_out >= target
            pos = jnp.where(any_, plsc.all_reduce_ffs(mask), 0)
            cum_at = plsc.load_gather(scr_a, [pos])
            cnt_at = plsc.load_gather(scr_b, [pos])
            upd = jnp.logical_and(found == 0, any_)
            tsel = jnp.where(upd, c * L + 15 - pos, tsel)
            ngt = jnp.where(upd, cum_at - cnt_at, ngt)
            found = jnp.where(upd, ones, found)
            return found, tsel, ngt, cum_out

        _, t, ngt, _ = lax.fori_loop(
            0, 16, sbody, (zeros, zeros, zeros, zeros))
        return t, ngt

    cp0 = pltpu.async_copy(probs_hbm.at[wid * 2], rowbuf, sem0)
    cp1 = pltpu.async_copy(probs_hbm.at[wid * 2 + 1], rowbuf2, sem1)
    for r, (rb, cp) in enumerate(((rowbuf, cp0), (rowbuf2, cp1))):
        row = wid * 2 + r
        cp.wait()

        t1, ngt1 = find_threshold(
            rb, _splat(K),
            lambda bits: lax.shift_right_logical(bits, 24),
            None)
        t2, _ = find_threshold(
            rb, _splat(K) - ngt1,
            lambda bits: jnp.bitwise_and(lax.shift_right_logical(bits, 16), 255),
            lambda bits: lax.shift_right_logical(bits, 24) == t1)
        pref = t1 * 256 + t2

        def cbody(j, off):
            bitss, msks, mis, incls = [], [], [], []
            for u in range(4):
                bits = rb[pl.ds(j * (4 * L) + u * L, L)]
                msk = lax.shift_right_logical(bits, 16) >= pref
                mi = msk.astype(jnp.int32)
                bitss.append(bits)
                msks.append(msk)
                mis.append(mi)
                incls.append(plsc.cumsum(mi))
            for u in range(4):
                hist_sort[pl.ds(u * L, L)] = incls[u]
            cnts = [plsc.load_gather(hist_sort, [_splat(u * L + 15)])
                    for u in range(4)]
            for u in range(4):
                posn = off + incls[u] - mis[u]
                plsc.store_scatter(key_a, [posn], bitss[u], mask=msks[u])
                plsc.store_scatter(idx_a, [posn], j * (4 * L) + u * L + lanes,
                                   mask=msks[u])
                off = off + cnts[u]
            return off
        m_cnt = lax.fori_loop(0, n_vec // 4, cbody, zeros)

        plsc.store_scatter(key_a, [m_cnt + lanes], zeros)
        plsc.store_scatter(idx_a, [m_cnt + lanes], zeros)

        b_vec = lax.shift_right_logical(m_cnt + 15, 4)
        b_sc = lax.squeeze(lax.slice(b_vec, (0,), (1,)), (0,))

        bufs = [(key_a, idx_a, key_b, idx_b), (key_b, idx_b, key_a, idx_a)]
        for p in range(8):
            shift = p * 4
            src_k, src_i, dst_k, dst_i = bufs[p % 2]
            for c in range(16):
                hist_sort[pl.ds(c * L, L)] = zeros

            def p1(j, carry):
                addr = lanes * b_sc + j
                kv = plsc.load_gather(src_k, [addr])
                dig = 15 - jnp.bitwise_and(
                    lax.shift_right_logical(kv, shift), 15)
                plsc.addupdate_scatter(hist_sort, [dig * L + lanes], ones)
                return carry
            lax.fori_loop(0, b_sc, p1, 0)

            def p2(c, carry):
                v = hist_sort[pl.ds(c * L, L)]
                incl = plsc.cumsum(v)
                hist_sort[pl.ds(c * L, L)] = incl - v + carry
                scr_a[...] = incl
                return carry + plsc.load_gather(scr_a, [_splat(15)])
            lax.fori_loop(0, 16, p2, zeros)

            def p3(j, carry):
                addr = lanes * b_sc + j
                kv = plsc.load_gather(src_k, [addr])
                iv = plsc.load_gather(src_i, [addr])
                dig = 15 - jnp.bitwise_and(
                    lax.shift_right_logical(kv, shift), 15)
                slot = dig * L + lanes
                off = plsc.load_gather(hist_sort, [slot])
                plsc.store_scatter(dst_k, [off], kv)
                plsc.store_scatter(dst_i, [off], iv)
                plsc.store_scatter(hist_sort, [slot], off + 1)
                return carry
            lax.fori_loop(0, b_sc, p3, 0)

        for c in range(K // L):
            valout[pl.ds(c * L, L)] = key_a[pl.ds(c * L, L)]
            idxout[pl.ds(c * L, L)] = idx_a[pl.ds(c * L, L)]
        pltpu.sync_copy(valout, vals_hbm.at[row])
        pltpu.sync_copy(idxout, idx_hbm.at[row])


def _topk_sc(probs_t):
    mesh = plsc.VectorSubcoreMesh(core_axis_name="c", subcore_axis_name="s")
    fn = functools.partial(
        pl.kernel,
        mesh=mesh,
        compiler_params=pltpu.CompilerParams(needs_layout_passes=False),
        out_type=[
            jax.ShapeDtypeStruct((NUM_E, K), jnp.int32),
            jax.ShapeDtypeStruct((NUM_E, K), jnp.int32),
        ],
        scratch_types=[
            pltpu.VMEM((NUM_T,), jnp.int32),
            pltpu.VMEM((NUM_T,), jnp.int32),
            pltpu.VMEM((4096,), jnp.int32),
            pltpu.VMEM((256,), jnp.int32),
            pltpu.VMEM((NUM_T + L,), jnp.int32),
            pltpu.VMEM((NUM_T + L,), jnp.int32),
            pltpu.VMEM((NUM_T + L,), jnp.int32),
            pltpu.VMEM((NUM_T + L,), jnp.int32),
            pltpu.VMEM((K,), jnp.int32),
            pltpu.VMEM((K,), jnp.int32),
            pltpu.VMEM((L,), jnp.int32),
            pltpu.VMEM((L,), jnp.int32),
            pltpu.SemaphoreType.DMA,
            pltpu.SemaphoreType.DMA,
        ],
    )(_topk_body)
    return fn(probs_t)


def kernel(x, W, b):
    probs_t = _probs_t(x, W, b)
    key_bits, idx = _topk_sc(lax.bitcast_convert_type(probs_t, jnp.int32))
    return lax.bitcast_convert_type(key_bits, jnp.float32), idx
